# baseline (speedup 1.0000x reference)
import functools

import jax
import jax.numpy as jnp
from jax import lax
from jax.experimental import pallas as pl
from jax.experimental.pallas import tpu as pltpu
from jax.experimental.pallas import tpu_sc as plsc

N_FIELDS = 26
VOCAB = 100000
EMB_DIM = 16
BATCH = 16384

GROUP = 128
TOTAL_ROWS = BATCH * N_FIELDS
TOTAL_GROUPS = TOTAL_ROWS // GROUP
NUM_WORKERS = 32
GROUPS_PER_W = TOTAL_GROUPS // NUM_WORKERS


def _sc_gather(flat_idx, flat_tables):
    mesh = plsc.VectorSubcoreMesh(core_axis_name="c", subcore_axis_name="s")

    @functools.partial(
        pl.kernel,
        out_type=jax.ShapeDtypeStruct((TOTAL_ROWS, EMB_DIM), jnp.float32),
        mesh=mesh,
        scratch_types=[
            pltpu.VMEM((GROUPS_PER_W, GROUP), jnp.int32),
            pltpu.VMEM((GROUP, EMB_DIM), jnp.float32),
            pltpu.SemaphoreType.DMA,
        ],
        compiler_params=pltpu.CompilerParams(use_tc_tiling_on_sc=False),
    )
    def run(idx_hbm, tab_hbm, out_hbm, idx_v, rows_v, sem):
        wid = lax.axis_index("s") * 2 + lax.axis_index("c")
        g0 = wid * GROUPS_PER_W
        pltpu.sync_copy(idx_hbm.at[pl.ds(g0, GROUPS_PER_W)], idx_v)

        def step(g, _):
            pltpu.async_copy(tab_hbm.at[idx_v.at[g]], rows_v, sem).wait()
            pltpu.sync_copy(rows_v, out_hbm.at[pl.ds((g0 + g) * GROUP, GROUP)])
            return _

        lax.fori_loop(0, GROUPS_PER_W, step, 0)

    return run(flat_idx, flat_tables)


def kernel(x, tables):
    flat_tables = tables.reshape(N_FIELDS * VOCAB, EMB_DIM)
    offs = (jnp.arange(N_FIELDS, dtype=jnp.int32) * VOCAB)[None, :]
    flat_idx = (x + offs).reshape(TOTAL_GROUPS, GROUP)
    out = _sc_gather(flat_idx, flat_tables)
    return out.reshape(BATCH, N_FIELDS, EMB_DIM)


# speedup vs baseline: 1.0430x; 1.0430x over previous
import functools

import jax
import jax.numpy as jnp
from jax import lax
from jax.experimental import pallas as pl
from jax.experimental.pallas import tpu as pltpu
from jax.experimental.pallas import tpu_sc as plsc

N_FIELDS = 26
VOCAB = 100000
EMB_DIM = 16
BATCH = 16384

GROUP = 128
TOTAL_ROWS = BATCH * N_FIELDS
TOTAL_GROUPS = TOTAL_ROWS // GROUP
NUM_WORKERS = 32
GROUPS_PER_W = TOTAL_GROUPS // NUM_WORKERS


CHUNK = 4
CHUNK_ROWS = CHUNK * GROUP
NCHUNKS = GROUPS_PER_W // CHUNK


def _sc_gather(flat_idx, flat_tables):
    mesh = plsc.VectorSubcoreMesh(core_axis_name="c", subcore_axis_name="s")

    @functools.partial(
        pl.kernel,
        out_type=jax.ShapeDtypeStruct((TOTAL_ROWS, EMB_DIM), jnp.float32),
        mesh=mesh,
        scratch_types=[
            pltpu.VMEM((GROUPS_PER_W, GROUP), jnp.int32),
            pltpu.VMEM((CHUNK_ROWS, EMB_DIM), jnp.float32),
            pltpu.VMEM((CHUNK_ROWS, EMB_DIM), jnp.float32),
            pltpu.SemaphoreType.DMA,
            pltpu.SemaphoreType.DMA,
            pltpu.SemaphoreType.DMA,
            pltpu.SemaphoreType.DMA,
        ],
        compiler_params=pltpu.CompilerParams(use_tc_tiling_on_sc=False),
    )
    def run(idx_hbm, tab_hbm, out_hbm, idx_v, buf0, buf1, sg0, sg1, so0, so1):
        wid = lax.axis_index("s") * 2 + lax.axis_index("c")
        g0 = wid * GROUPS_PER_W
        row0 = g0 * GROUP
        pltpu.sync_copy(idx_hbm.at[pl.ds(g0, GROUPS_PER_W)], idx_v)

        def fire(c, buf, sem):
            h = []
            for j in range(CHUNK):
                h.append(pltpu.async_copy(
                    tab_hbm.at[idx_v.at[c * CHUNK + j]],
                    buf.at[pl.ds(j * GROUP, GROUP)], sem))
            return h

        def out_fire(c, buf, sem):
            return pltpu.async_copy(
                buf, out_hbm.at[pl.ds(row0 + c * CHUNK_ROWS, CHUNK_ROWS)], sem)

        def out_wait(c, buf, sem):
            pltpu.make_async_copy(
                buf, out_hbm.at[pl.ds(row0 + c * CHUNK_ROWS, CHUNK_ROWS)], sem
            ).wait()

        def gather_wait(c, buf, sem):
            for j in range(CHUNK):
                pltpu.make_async_copy(
                    tab_hbm.at[idx_v.at[c * CHUNK + j]],
                    buf.at[pl.ds(j * GROUP, GROUP)], sem).wait()

        fire(0, buf0, sg0)

        @pl.loop(0, NCHUNKS, step=2)
        def _body(c):
            @pl.when(c > 0)
            def _():
                out_wait(c - 1, buf1, so1)

            fire(c + 1, buf1, sg1)
            gather_wait(c, buf0, sg0)
            out_fire(c, buf0, so0)
            gather_wait(c + 1, buf1, sg1)
            out_fire(c + 1, buf1, so1)
            out_wait(c, buf0, so0)

            @pl.when(c + 2 < NCHUNKS)
            def _():
                fire(c + 2, buf0, sg0)

        out_wait(NCHUNKS - 1, buf1, so1)

    return run(flat_idx, flat_tables)


def kernel(x, tables):
    flat_tables = tables.reshape(N_FIELDS * VOCAB, EMB_DIM)
    offs = (jnp.arange(N_FIELDS, dtype=jnp.int32) * VOCAB)[None, :]
    flat_idx = (x + offs).reshape(TOTAL_GROUPS, GROUP)
    out = _sc_gather(flat_idx, flat_tables)
    return out.reshape(BATCH, N_FIELDS, EMB_DIM)


# speedup vs baseline: 6.8660x; 6.5828x over previous
import functools

import jax
import jax.numpy as jnp
from jax import lax
from jax.experimental import pallas as pl
from jax.experimental.pallas import tpu as pltpu
from jax.experimental.pallas import tpu_sc as plsc

N_FIELDS = 26
VOCAB = 100000
EMB_DIM = 16
BATCH = 16384

NUM_WORKERS = 32
NUM_PLANES = N_FIELDS * EMB_DIM
PLANES_PER_W = NUM_PLANES // NUM_WORKERS
OUT_CHUNK = 8192


def _sc_lookup(x_t, tab_t):
    mesh = plsc.VectorSubcoreMesh(core_axis_name="c", subcore_axis_name="s")

    @functools.partial(
        pl.kernel,
        out_type=jax.ShapeDtypeStruct((N_FIELDS, EMB_DIM, BATCH), jnp.float32),
        mesh=mesh,
        scratch_types=[
            pltpu.VMEM((VOCAB,), jnp.float32),
            pltpu.VMEM((BATCH,), jnp.int32),
            pltpu.VMEM((OUT_CHUNK,), jnp.float32),
        ],
        compiler_params=pltpu.CompilerParams(needs_layout_passes=False),
    )
    def run(x_hbm, tab_hbm, out_hbm, plane_v, idx_v, outc_v):
        wid = lax.axis_index("s") * 2 + lax.axis_index("c")
        p0 = wid * PLANES_PER_W

        def do_plane(k, _):
            p = p0 + k
            f = p // EMB_DIM
            d = p % EMB_DIM
            pltpu.sync_copy(x_hbm.at[f], idx_v)
            pltpu.sync_copy(tab_hbm.at[f, d], plane_v)

            def do_chunk(h, _):
                def gather16(i, _):
                    vidx = idx_v[pl.ds(h * OUT_CHUNK + i * 16, 16)]
                    outc_v[pl.ds(i * 16, 16)] = plsc.load_gather(
                        plane_v, [vidx])
                    return _

                lax.fori_loop(0, OUT_CHUNK // 16, gather16, 0)
                pltpu.sync_copy(
                    outc_v, out_hbm.at[f, d, pl.ds(h * OUT_CHUNK, OUT_CHUNK)])
                return _

            lax.fori_loop(0, BATCH // OUT_CHUNK, do_chunk, 0)
            return _

        lax.fori_loop(0, PLANES_PER_W, do_plane, 0)

    return run(x_t, tab_t)


def kernel(x, tables):
    tab_t = tables.transpose(0, 2, 1)
    x_t = x.T
    out_t = _sc_lookup(x_t, tab_t)
    return out_t.transpose(2, 0, 1)


# speedup vs baseline: 15.2556x; 2.2219x over previous
import functools

import jax
import jax.numpy as jnp
from jax import lax
from jax.experimental import pallas as pl
from jax.experimental.pallas import tpu as pltpu
from jax.experimental.pallas import tpu_sc as plsc

N_FIELDS = 26
VOCAB = 100000
EMB_DIM = 16
BATCH = 16384

NUM_WORKERS = 32
NUM_PLANES = N_FIELDS * EMB_DIM
PLANES_PER_W = NUM_PLANES // NUM_WORKERS
OUT_CHUNK = 4096
NCHUNK = BATCH // OUT_CHUNK


def _sc_lookup(x_t, tab_t):
    mesh = plsc.VectorSubcoreMesh(core_axis_name="c", subcore_axis_name="s")

    @functools.partial(
        pl.kernel,
        out_type=jax.ShapeDtypeStruct((N_FIELDS, EMB_DIM, BATCH), jnp.float32),
        mesh=mesh,
        scratch_types=[
            pltpu.VMEM((VOCAB,), jnp.float32),
            pltpu.VMEM((BATCH,), jnp.int32),
            pltpu.VMEM((OUT_CHUNK,), jnp.float32),
            pltpu.VMEM((OUT_CHUNK,), jnp.float32),
            pltpu.SemaphoreType.DMA,
            pltpu.SemaphoreType.DMA,
            pltpu.SemaphoreType.DMA,
            pltpu.SemaphoreType.DMA,
        ],
        compiler_params=pltpu.CompilerParams(needs_layout_passes=False),
    )
    def run(x_hbm, tab_hbm, out_hbm, plane_v, idx_v, oc0, oc1, sp, si, so0, so1):
        wid = lax.axis_index("s") * 2 + lax.axis_index("c")
        p0 = wid * PLANES_PER_W
        ocs = (oc0, oc1)
        sos = (so0, so1)

        def do_plane(p, first_out):
            f = p // EMB_DIM
            d = p % EMB_DIM
            pltpu.async_copy(tab_hbm.at[f, d], plane_v, sp)
            pltpu.make_async_copy(tab_hbm.at[f, d], plane_v, sp).wait()

            for h in range(NCHUNK):
                ob = ocs[h % 2]
                sem = sos[h % 2]

                def drain(ob=ob, sem=sem):
                    pltpu.make_async_copy(
                        ob, out_hbm.at[0, 0, pl.ds(0, OUT_CHUNK)], sem).wait()

                if h < 2:
                    pl.when(jnp.logical_not(first_out))(drain)
                else:
                    drain()

                @plsc.parallel_loop(0, OUT_CHUNK // 16, unroll=8)
                def _(i, h=h, ob=ob):
                    vidx = idx_v[pl.ds(h * OUT_CHUNK + i * 16, 16)]
                    ob[pl.ds(i * 16, 16)] = plsc.load_gather(plane_v, [vidx])

                pltpu.async_copy(
                    ob, out_hbm.at[f, d, pl.ds(h * OUT_CHUNK, OUT_CHUNK)], sem)
            return jnp.bool_(False)

        f0 = p0 // EMB_DIM
        k_split = jnp.minimum(PLANES_PER_W, (f0 + 1) * EMB_DIM - p0)

        pltpu.sync_copy(x_hbm.at[f0], idx_v)
        first = lax.fori_loop(
            0, k_split, lambda k, fo: do_plane(p0 + k, fo), jnp.bool_(True))

        @pl.when(k_split < PLANES_PER_W)
        def _():
            pltpu.sync_copy(x_hbm.at[f0 + 1], idx_v)
            lax.fori_loop(k_split, PLANES_PER_W,
                          lambda k, fo: do_plane(p0 + k, fo), first)

        pltpu.make_async_copy(
            oc0, out_hbm.at[0, 0, pl.ds(0, OUT_CHUNK)], so0).wait()
        pltpu.make_async_copy(
            oc1, out_hbm.at[0, 0, pl.ds(0, OUT_CHUNK)], so1).wait()

    return run(x_t, tab_t)


def kernel(x, tables):
    tab_t = tables.transpose(0, 2, 1)
    x_t = x.T
    out_t = _sc_lookup(x_t, tab_t)
    return out_t.transpose(2, 0, 1)
